# Initial kernel scaffold; baseline (speedup 1.0000x reference)
import jax
import jax.numpy as jnp
from jax.experimental import pallas as pl


def kernel(x, edge_index, batch, Wr0, Wa0, b0, g0, be0, Wr1, Wa1, b1, g1, be1, Ws1, bs1, Ws2, bs2, Wh1, bh1, Wh2, bh2, Wh3, bh3):
    raise NotImplementedError("write your pallas kernel here")


# speedup vs baseline: 5.6072x; 5.6072x over previous
import functools

import jax
import jax.numpy as jnp
from jax import lax
from jax.experimental import pallas as pl
from jax.experimental.pallas import tpu as pltpu
from jax.experimental.pallas import tpu_sc as plsc

N_NODES = 10000
N_EDGES = 320000
D_FEAT = 128
BATCH_SIZE = 100
NODES_PER_GRAPH = 100

NC = 2
NS = 16
NW = NC * NS
EPW = N_EDGES // NW
CHUNK = 80
NCHUNKS = EPW // CHUNK
N_PAD = 10240
RPT = N_PAD // NS
RCH = RPT // CHUNK


def _sc_agg_body(with_deg, *refs):
    if with_deg:
        (h_hbm, src_hbm, dst_hbm, zfeat_hbm, zdeg_hbm, ones_hbm,
         agg_out, deg_out,
         agg_sh, deg_sh, rows_v, sidx_v, didx_v, ones_v, dstage_v,
         sem) = refs
    else:
        (h_hbm, src_hbm, dst_hbm, zfeat_hbm,
         agg_out,
         agg_sh, rows_v, sidx_v, didx_v, sem) = refs

    cid = lax.axis_index("c")
    sid = lax.axis_index("s")
    wid = sid * NC + cid

    pltpu.sync_copy(zfeat_hbm, rows_v)
    for k in range(RCH):
        pltpu.sync_copy(rows_v, agg_sh.at[pl.ds(sid * RPT + k * CHUNK, CHUNK)])
    if with_deg:
        pltpu.sync_copy(zdeg_hbm, dstage_v)
        pltpu.sync_copy(dstage_v, deg_sh.at[pl.ds(sid * RPT, RPT)])
        pltpu.sync_copy(ones_hbm, ones_v)
    plsc.subcore_barrier()

    def body(i, carry):
        base = wid * EPW + i * CHUNK
        pltpu.sync_copy(src_hbm.at[pl.ds(base, CHUNK)], sidx_v)
        pltpu.sync_copy(dst_hbm.at[pl.ds(base, CHUNK)], didx_v)
        pltpu.async_copy(h_hbm.at[sidx_v], rows_v, sem).wait()
        pltpu.sync_copy(rows_v, agg_sh.at[didx_v], add=True)
        if with_deg:
            pltpu.sync_copy(ones_v, deg_sh.at[didx_v], add=True)
        return carry

    lax.fori_loop(0, NCHUNKS, body, 0)
    plsc.subcore_barrier()

    for k in range(RCH):
        pltpu.sync_copy(agg_sh.at[pl.ds(sid * RPT + k * CHUNK, CHUNK)], rows_v)
        pltpu.sync_copy(rows_v, agg_out.at[cid, pl.ds(sid * RPT + k * CHUNK, CHUNK)])
    if with_deg:
        pltpu.sync_copy(deg_sh.at[pl.ds(sid * RPT, RPT)], dstage_v)
        pltpu.sync_copy(dstage_v, deg_out.at[pl.ds(cid * N_PAD + sid * RPT, RPT)])


def _sc_aggregate(h, src, dst, with_deg):
    mesh = plsc.VectorSubcoreMesh(core_axis_name="c", subcore_axis_name="s",
                                  num_cores=NC, num_subcores=NS)
    zfeat = jnp.zeros((CHUNK, D_FEAT), jnp.float32)
    if with_deg:
        out_type = (jax.ShapeDtypeStruct((NC, N_PAD, D_FEAT), jnp.float32),
                    jax.ShapeDtypeStruct((NC * N_PAD,), jnp.float32))
        scratch = [
            pltpu.VMEM_SHARED((N_PAD, D_FEAT), jnp.float32),
            pltpu.VMEM_SHARED((N_PAD,), jnp.float32),
            pltpu.VMEM((CHUNK, D_FEAT), jnp.float32),
            pltpu.VMEM((CHUNK,), jnp.int32),
            pltpu.VMEM((CHUNK,), jnp.int32),
            pltpu.VMEM((CHUNK,), jnp.float32),
            pltpu.VMEM((RPT,), jnp.float32),
            pltpu.SemaphoreType.DMA,
        ]
        zdeg = jnp.zeros((RPT,), jnp.float32)
        ones = jnp.ones((CHUNK,), jnp.float32)
        fn = pl.kernel(functools.partial(_sc_agg_body, True),
                       out_type=out_type, mesh=mesh, scratch_types=scratch)
        return fn(h, src, dst, zfeat, zdeg, ones)
    else:
        out_type = jax.ShapeDtypeStruct((NC, N_PAD, D_FEAT), jnp.float32)
        scratch = [
            pltpu.VMEM_SHARED((N_PAD, D_FEAT), jnp.float32),
            pltpu.VMEM((CHUNK, D_FEAT), jnp.float32),
            pltpu.VMEM((CHUNK,), jnp.int32),
            pltpu.VMEM((CHUNK,), jnp.int32),
            pltpu.SemaphoreType.DMA,
        ]
        fn = pl.kernel(functools.partial(_sc_agg_body, False),
                       out_type=out_type, mesh=mesh, scratch_types=scratch)
        return fn(h, src, dst, zfeat)


def _dense1_body(x_ref, agg_ref, degA_ref, degB_ref, Wr_ref, Wa_ref, b_ref, g_ref,
                 be_ref, o_ref):
    agg = agg_ref[0, :N_NODES] + agg_ref[1, :N_NODES]
    deg = degA_ref[:N_NODES] + degB_ref[:N_NODES]
    mean = agg / jnp.maximum(deg, 1.0)
    c = (jnp.dot(x_ref[...], Wr_ref[...], preferred_element_type=jnp.float32)
         + jnp.dot(mean, Wa_ref[...], preferred_element_type=jnp.float32)
         + b_ref[...])
    mu = jnp.mean(c, axis=0, keepdims=True)
    var = jnp.mean((c - mu) * (c - mu), axis=0, keepdims=True)
    h = (c - mu) * lax.rsqrt(var + 1e-5) * g_ref[...] + be_ref[...]
    o_ref[...] = jnp.maximum(h, 0.0)


def _dense1(x, agg, degA, degB, Wr, Wa, b, g, be):
    return pl.pallas_call(
        _dense1_body,
        out_shape=jax.ShapeDtypeStruct((N_NODES, D_FEAT), jnp.float32),
    )(x, agg, degA, degB, Wr, Wa, b.reshape(1, -1),
      g.reshape(1, -1), be.reshape(1, -1))


def _dense2_body(h_ref, agg_ref, degA_ref, degB_ref, Wr_ref, Wa_ref, b_ref, g_ref,
                 be_ref, Ws1_ref, bs1_ref, Ws2_ref, bs2_ref, Wh1_ref,
                 bh1_ref, Wh2_ref, bh2_ref, Wh3_ref, bh3_ref, o_ref):
    agg = agg_ref[0, :N_NODES] + agg_ref[1, :N_NODES]
    deg = degA_ref[:N_NODES] + degB_ref[:N_NODES]
    mean = agg / jnp.maximum(deg, 1.0)
    c = (jnp.dot(h_ref[...], Wr_ref[...], preferred_element_type=jnp.float32)
         + jnp.dot(mean, Wa_ref[...], preferred_element_type=jnp.float32)
         + b_ref[...])
    mu = jnp.mean(c, axis=0, keepdims=True)
    var = jnp.mean((c - mu) * (c - mu), axis=0, keepdims=True)
    h = (c - mu) * lax.rsqrt(var + 1e-5) * g_ref[...] + be_ref[...]
    h = jnp.maximum(h, 0.0)

    hg = jnp.mean(h.reshape(BATCH_SIZE, NODES_PER_GRAPH, D_FEAT), axis=1)

    t = jnp.maximum(hg, 0.0)
    t = jnp.dot(t, Ws1_ref[...], preferred_element_type=jnp.float32) + bs1_ref[...]
    t = jnp.dot(t, Ws2_ref[...], preferred_element_type=jnp.float32) + bs2_ref[...]
    t = jnp.maximum(t, 0.0)
    t = jnp.maximum(jnp.dot(t, Wh1_ref[...], preferred_element_type=jnp.float32) + bh1_ref[...], 0.0)
    t = jnp.maximum(jnp.dot(t, Wh2_ref[...], preferred_element_type=jnp.float32) + bh2_ref[...], 0.0)
    o_ref[...] = jnp.dot(t, Wh3_ref[...], preferred_element_type=jnp.float32) + bh3_ref[...]


def _dense2(h, agg, degA, degB, Wr, Wa, b, g, be, Ws1, bs1, Ws2, bs2,
            Wh1, bh1, Wh2, bh2, Wh3, bh3):
    return pl.pallas_call(
        _dense2_body,
        out_shape=jax.ShapeDtypeStruct((BATCH_SIZE, 1), jnp.float32),
    )(h, agg, degA, degB, Wr, Wa, b.reshape(1, -1), g.reshape(1, -1),
      be.reshape(1, -1), Ws1, bs1.reshape(1, -1), Ws2, bs2.reshape(1, -1),
      Wh1, bh1.reshape(1, -1), Wh2, bh2.reshape(1, -1), Wh3,
      bh3.reshape(1, -1))


def kernel(x, edge_index, batch, Wr0, Wa0, b0, g0, be0, Wr1, Wa1, b1, g1,
           be1, Ws1, bs1, Ws2, bs2, Wh1, bh1, Wh2, bh2, Wh3, bh3):
    src = edge_index[0]
    dst = edge_index[1]
    agg0, deg_flat = _sc_aggregate(x, src, dst, with_deg=True)
    degA = deg_flat[:N_PAD].reshape(N_PAD, 1)
    degB = deg_flat[N_PAD:].reshape(N_PAD, 1)
    h1 = _dense1(x, agg0, degA, degB, Wr0, Wa0, b0, g0, be0)
    agg1 = _sc_aggregate(h1, src, dst, with_deg=False)
    return _dense2(h1, agg1, degA, degB, Wr1, Wa1, b1, g1, be1, Ws1, bs1,
                   Ws2, bs2, Wh1, bh1, Wh2, bh2, Wh3, bh3)


# speedup vs baseline: 12.5086x; 2.2308x over previous
import functools

import jax
import jax.numpy as jnp
from jax import lax
from jax.experimental import pallas as pl
from jax.experimental.pallas import tpu as pltpu
from jax.experimental.pallas import tpu_sc as plsc

N_NODES = 10000
N_EDGES = 320000
D_FEAT = 128
BATCH_SIZE = 100
NODES_PER_GRAPH = 100

NC = 2
NS = 16
NW = NC * NS
EPW = N_EDGES // NW
CHUNK = 80
NCHUNKS = EPW // CHUNK
N_PAD = 10240
RPT = N_PAD // NS
RCH = RPT // CHUNK


def _sc_agg_body(with_deg, *refs):
    if with_deg:
        (h_hbm, src_hbm, dst_hbm, zfeat_hbm, zdeg_hbm, ones_hbm,
         agg_out, deg_out,
         agg_sh, deg_sh, rows0_v, rows1_v, sidx_v, didx_v, ones_v,
         dstage_v, sem0, sem1) = refs
    else:
        (h_hbm, src_hbm, dst_hbm, zfeat_hbm,
         agg_out,
         agg_sh, rows0_v, rows1_v, sidx_v, didx_v, sem0, sem1) = refs

    cid = lax.axis_index("c")
    sid = lax.axis_index("s")
    wid = sid * NC + cid

    pltpu.sync_copy(src_hbm.at[pl.ds(wid * EPW, EPW)], sidx_v)
    pltpu.sync_copy(dst_hbm.at[pl.ds(wid * EPW, EPW)], didx_v)

    pltpu.sync_copy(zfeat_hbm, rows0_v)
    for k in range(RCH):
        pltpu.sync_copy(rows0_v, agg_sh.at[pl.ds(sid * RPT + k * CHUNK, CHUNK)])
    if with_deg:
        pltpu.sync_copy(zdeg_hbm, dstage_v)
        pltpu.sync_copy(dstage_v, deg_sh.at[pl.ds(sid * RPT, RPT)])
        pltpu.sync_copy(ones_hbm, ones_v)
    plsc.subcore_barrier()

    def gather(i, rows, sem):
        pltpu.async_copy(
            h_hbm.at[sidx_v.at[pl.ds(i * CHUNK, CHUNK)]], rows, sem)

    def gwait(rows, sem):
        pltpu.make_async_copy(
            h_hbm.at[sidx_v.at[pl.ds(0, CHUNK)]], rows, sem).wait()

    def scatter(i, rows):
        idx = didx_v.at[pl.ds(i * CHUNK, CHUNK)]
        pltpu.sync_copy(rows, agg_sh.at[idx], add=True)
        if with_deg:
            pltpu.sync_copy(ones_v, deg_sh.at[idx], add=True)

    gather(0, rows0_v, sem0)

    def pair(j, carry):
        i = 2 * j
        gather(i + 1, rows1_v, sem1)
        gwait(rows0_v, sem0)
        scatter(i, rows0_v)
        gather(i + 2, rows0_v, sem0)
        gwait(rows1_v, sem1)
        scatter(i + 1, rows1_v)
        return carry

    lax.fori_loop(0, (NCHUNKS - 1) // 2, pair, 0)
    gwait(rows0_v, sem0)
    scatter(NCHUNKS - 1, rows0_v)
    plsc.subcore_barrier()

    for k in range(RCH):
        pltpu.sync_copy(agg_sh.at[pl.ds(sid * RPT + k * CHUNK, CHUNK)], rows0_v)
        pltpu.sync_copy(rows0_v, agg_out.at[cid, pl.ds(sid * RPT + k * CHUNK, CHUNK)])
    if with_deg:
        pltpu.sync_copy(deg_sh.at[pl.ds(sid * RPT, RPT)], dstage_v)
        pltpu.sync_copy(dstage_v, deg_out.at[pl.ds(cid * N_PAD + sid * RPT, RPT)])


def _sc_aggregate(h, src, dst, with_deg):
    mesh = plsc.VectorSubcoreMesh(core_axis_name="c", subcore_axis_name="s",
                                  num_cores=NC, num_subcores=NS)
    zfeat = jnp.zeros((CHUNK, D_FEAT), jnp.float32)
    if with_deg:
        out_type = (jax.ShapeDtypeStruct((NC, N_PAD, D_FEAT), jnp.float32),
                    jax.ShapeDtypeStruct((NC * N_PAD,), jnp.float32))
        scratch = [
            pltpu.VMEM_SHARED((N_PAD, D_FEAT), jnp.float32),
            pltpu.VMEM_SHARED((N_PAD,), jnp.float32),
            pltpu.VMEM((CHUNK, D_FEAT), jnp.float32),
            pltpu.VMEM((CHUNK, D_FEAT), jnp.float32),
            pltpu.VMEM((EPW,), jnp.int32),
            pltpu.VMEM((EPW,), jnp.int32),
            pltpu.VMEM((CHUNK,), jnp.float32),
            pltpu.VMEM((RPT,), jnp.float32),
            pltpu.SemaphoreType.DMA,
            pltpu.SemaphoreType.DMA,
        ]
        zdeg = jnp.zeros((RPT,), jnp.float32)
        ones = jnp.ones((CHUNK,), jnp.float32)
        fn = pl.kernel(functools.partial(_sc_agg_body, True),
                       out_type=out_type, mesh=mesh, scratch_types=scratch)
        return fn(h, src, dst, zfeat, zdeg, ones)
    else:
        out_type = jax.ShapeDtypeStruct((NC, N_PAD, D_FEAT), jnp.float32)
        scratch = [
            pltpu.VMEM_SHARED((N_PAD, D_FEAT), jnp.float32),
            pltpu.VMEM((CHUNK, D_FEAT), jnp.float32),
            pltpu.VMEM((CHUNK, D_FEAT), jnp.float32),
            pltpu.VMEM((EPW,), jnp.int32),
            pltpu.VMEM((EPW,), jnp.int32),
            pltpu.SemaphoreType.DMA,
            pltpu.SemaphoreType.DMA,
        ]
        fn = pl.kernel(functools.partial(_sc_agg_body, False),
                       out_type=out_type, mesh=mesh, scratch_types=scratch)
        return fn(h, src, dst, zfeat)


def _dense1_body(x_ref, agg_ref, degA_ref, degB_ref, Wr_ref, Wa_ref, b_ref, g_ref,
                 be_ref, o_ref):
    agg = agg_ref[0, :N_NODES] + agg_ref[1, :N_NODES]
    deg = degA_ref[:N_NODES] + degB_ref[:N_NODES]
    mean = agg / jnp.maximum(deg, 1.0)
    c = (jnp.dot(x_ref[...], Wr_ref[...], preferred_element_type=jnp.float32)
         + jnp.dot(mean, Wa_ref[...], preferred_element_type=jnp.float32)
         + b_ref[...])
    mu = jnp.mean(c, axis=0, keepdims=True)
    var = jnp.mean((c - mu) * (c - mu), axis=0, keepdims=True)
    h = (c - mu) * lax.rsqrt(var + 1e-5) * g_ref[...] + be_ref[...]
    o_ref[...] = jnp.maximum(h, 0.0)


def _dense1(x, agg, degA, degB, Wr, Wa, b, g, be):
    return pl.pallas_call(
        _dense1_body,
        out_shape=jax.ShapeDtypeStruct((N_NODES, D_FEAT), jnp.float32),
    )(x, agg, degA, degB, Wr, Wa, b.reshape(1, -1),
      g.reshape(1, -1), be.reshape(1, -1))


def _dense2_body(h_ref, agg_ref, degA_ref, degB_ref, Wr_ref, Wa_ref, b_ref, g_ref,
                 be_ref, Ws1_ref, bs1_ref, Ws2_ref, bs2_ref, Wh1_ref,
                 bh1_ref, Wh2_ref, bh2_ref, Wh3_ref, bh3_ref, o_ref):
    agg = agg_ref[0, :N_NODES] + agg_ref[1, :N_NODES]
    deg = degA_ref[:N_NODES] + degB_ref[:N_NODES]
    mean = agg / jnp.maximum(deg, 1.0)
    c = (jnp.dot(h_ref[...], Wr_ref[...], preferred_element_type=jnp.float32)
         + jnp.dot(mean, Wa_ref[...], preferred_element_type=jnp.float32)
         + b_ref[...])
    mu = jnp.mean(c, axis=0, keepdims=True)
    var = jnp.mean((c - mu) * (c - mu), axis=0, keepdims=True)
    h = (c - mu) * lax.rsqrt(var + 1e-5) * g_ref[...] + be_ref[...]
    h = jnp.maximum(h, 0.0)

    hg = jnp.mean(h.reshape(BATCH_SIZE, NODES_PER_GRAPH, D_FEAT), axis=1)

    t = jnp.maximum(hg, 0.0)
    t = jnp.dot(t, Ws1_ref[...], preferred_element_type=jnp.float32) + bs1_ref[...]
    t = jnp.dot(t, Ws2_ref[...], preferred_element_type=jnp.float32) + bs2_ref[...]
    t = jnp.maximum(t, 0.0)
    t = jnp.maximum(jnp.dot(t, Wh1_ref[...], preferred_element_type=jnp.float32) + bh1_ref[...], 0.0)
    t = jnp.maximum(jnp.dot(t, Wh2_ref[...], preferred_element_type=jnp.float32) + bh2_ref[...], 0.0)
    o_ref[...] = jnp.dot(t, Wh3_ref[...], preferred_element_type=jnp.float32) + bh3_ref[...]


def _dense2(h, agg, degA, degB, Wr, Wa, b, g, be, Ws1, bs1, Ws2, bs2,
            Wh1, bh1, Wh2, bh2, Wh3, bh3):
    return pl.pallas_call(
        _dense2_body,
        out_shape=jax.ShapeDtypeStruct((BATCH_SIZE, 1), jnp.float32),
    )(h, agg, degA, degB, Wr, Wa, b.reshape(1, -1), g.reshape(1, -1),
      be.reshape(1, -1), Ws1, bs1.reshape(1, -1), Ws2, bs2.reshape(1, -1),
      Wh1, bh1.reshape(1, -1), Wh2, bh2.reshape(1, -1), Wh3,
      bh3.reshape(1, -1))


def kernel(x, edge_index, batch, Wr0, Wa0, b0, g0, be0, Wr1, Wa1, b1, g1,
           be1, Ws1, bs1, Ws2, bs2, Wh1, bh1, Wh2, bh2, Wh3, bh3):
    src = edge_index[0]
    dst = edge_index[1]
    agg0, deg_flat = _sc_aggregate(x, src, dst, with_deg=True)
    degA = deg_flat[:N_PAD].reshape(N_PAD, 1)
    degB = deg_flat[N_PAD:].reshape(N_PAD, 1)
    h1 = _dense1(x, agg0, degA, degB, Wr0, Wa0, b0, g0, be0)
    agg1 = _sc_aggregate(h1, src, dst, with_deg=False)
    return _dense2(h1, agg1, degA, degB, Wr1, Wa1, b1, g1, be1, Ws1, bs1,
                   Ws2, bs2, Wh1, bh1, Wh2, bh2, Wh3, bh3)
